# baseline (speedup 1.0000x reference)
import jax
import jax.numpy as jnp
from jax.experimental import pallas as pl

N_P = 8
N_S = 64
N_A = N_P + N_S
COMM_RANGE = 0.3
HID = 256
BG = 16

_EXACT = jax.lax.Precision.HIGHEST


def _iota2(shape, dim):
    return jax.lax.broadcasted_iota(jnp.int32, shape, dim)


def _body(obs_ref, ph_ref, s_ref, wq_ref, wk_ref, wv_ref, fcw_ref, fcb_ref,
          h_out_ref, alpha_ref):
    ph = ph_ref[...]
    s = s_ref[...]
    R = ph.shape[0]
    C = s.shape[0]

    ob = obs_ref[...]
    L = N_A * 8
    sel_px = (_iota2((L, N_P), 0) == 8 * _iota2((L, N_P), 1)).astype(jnp.float32)
    sel_py = (_iota2((L, N_P), 0) == 8 * _iota2((L, N_P), 1) + 1).astype(jnp.float32)
    sel_sx = (_iota2((L, N_S), 0) == 8 * (_iota2((L, N_S), 1) + N_P)).astype(jnp.float32)
    sel_sy = (_iota2((L, N_S), 0) == 8 * (_iota2((L, N_S), 1) + N_P) + 1).astype(jnp.float32)
    PX = jnp.dot(ob, sel_px, precision=_EXACT)
    PY = jnp.dot(ob, sel_py, precision=_EXACT)
    SX = jnp.dot(ob, sel_sx, precision=_EXACT)
    SY = jnp.dot(ob, sel_sy, precision=_EXACT)

    eb = (_iota2((R, BG), 0) // N_P == _iota2((R, BG), 1)).astype(jnp.float32)
    lane_eq = _iota2((R, N_P), 1) == _iota2((R, N_P), 0) % N_P
    px = jnp.sum(jnp.where(lane_eq, jnp.dot(eb, PX, precision=_EXACT), 0.0),
                 axis=1, keepdims=True)
    py = jnp.sum(jnp.where(lane_eq, jnp.dot(eb, PY, precision=_EXACT), 0.0),
                 axis=1, keepdims=True)

    keep = _iota2((BG, C), 1) // N_S == _iota2((BG, C), 0)
    sx = jnp.sum(jnp.where(keep, jnp.concatenate([SX] * BG, axis=1), 0.0),
                 axis=0, keepdims=True)
    sy = jnp.sum(jnp.where(keep, jnp.concatenate([SY] * BG, axis=1), 0.0),
                 axis=0, keepdims=True)

    sb = s.astype(jnp.bfloat16)
    q = jnp.dot(ph.astype(jnp.bfloat16), wq_ref[...].astype(jnp.bfloat16),
                preferred_element_type=jnp.float32)
    k = jnp.dot(sb, wk_ref[...].astype(jnp.bfloat16),
                preferred_element_type=jnp.float32)
    v = jnp.dot(sb, wv_ref[...].astype(jnp.bfloat16),
                preferred_element_type=jnp.float32)

    e = jax.lax.dot_general(q.astype(jnp.bfloat16), k.astype(jnp.bfloat16),
                            (((1,), (1,)), ((), ())),
                            preferred_element_type=jnp.float32)
    e = e * (1.0 / jnp.sqrt(jnp.float32(HID)))

    dx = jnp.abs(px - sx)
    dy = jnp.abs(py - sy)
    near = jnp.maximum(dx, dy) <= COMM_RANGE
    mask = near & (_iota2((R, C), 0) // N_P == _iota2((R, C), 1) // N_S)

    e = jnp.where(mask, e, -1e30)
    m = jnp.max(e, axis=1, keepdims=True)
    ex = jnp.exp(e - m)
    a = ex / jnp.sum(ex, axis=1, keepdims=True)
    a = jnp.where(mask, a, 0.0)
    has_vis = jnp.any(mask, axis=1, keepdims=True)

    attn = jnp.dot(a.astype(jnp.bfloat16), v.astype(jnp.bfloat16),
                   preferred_element_type=jnp.float32)
    h = jnp.where(has_vis, attn, ph)
    h_out_ref[...] = jnp.dot(h.astype(jnp.bfloat16),
                             fcw_ref[...].astype(jnp.bfloat16),
                             preferred_element_type=jnp.float32) + fcb_ref[...]

    sel = (_iota2((C, N_S), 0) % N_S == _iota2((C, N_S), 1)).astype(jnp.float32)
    alpha_ref[...] = jnp.dot(a, sel, precision=_EXACT,
                             preferred_element_type=jnp.float32)


def kernel(obs, p_hidden, s_hidden, batch_size, Wq, Wk, Wv, fc_W, fc_b):
    B = p_hidden.shape[0] // N_P
    G = B // BG
    obs32 = obs.reshape(B, N_A * 8)
    s_flat = s_hidden.reshape(B * N_S, HID)
    fc_b2 = fc_b.reshape(1, HID)

    const2d = pl.BlockSpec((HID, HID), lambda g: (0, 0))
    h_out, alpha = pl.pallas_call(
        _body,
        grid=(G,),
        in_specs=[
            pl.BlockSpec((BG, N_A * 8), lambda g: (g, 0)),
            pl.BlockSpec((BG * N_P, HID), lambda g: (g, 0)),
            pl.BlockSpec((BG * N_S, HID), lambda g: (g, 0)),
            const2d, const2d, const2d, const2d,
            pl.BlockSpec((1, HID), lambda g: (0, 0)),
        ],
        out_specs=[
            pl.BlockSpec((BG * N_P, HID), lambda g: (g, 0)),
            pl.BlockSpec((BG * N_P, N_S), lambda g: (g, 0)),
        ],
        out_shape=[
            jax.ShapeDtypeStruct((B * N_P, HID), jnp.float32),
            jax.ShapeDtypeStruct((B * N_P, N_S), jnp.float32),
        ],
    )(obs32, p_hidden, s_flat, Wq, Wk, Wv, fc_W, fc_b2)
    return h_out.reshape(B, N_P, HID), alpha.reshape(B, N_P, N_S)
